# Initial kernel scaffold; baseline (speedup 1.0000x reference)
import jax
import jax.numpy as jnp
from jax.experimental import pallas as pl


def kernel(features, radial_basis_function_coefficients, ab_p_to_a, ab_p_to_b, Ys, radii, cg, mus, norm_coef):
    raise NotImplementedError("write your pallas kernel here")


# speedup vs baseline: 1.5634x; 1.5634x over previous
import functools

import jax
import jax.numpy as jnp
from jax import lax
from jax.experimental import pallas as pl
from jax.experimental.pallas import tpu as pltpu
from jax.experimental.pallas import tpu_sc as plsc

D_IN = 32
D_OUT = 32
D_SH = 4
D_RAD = 8

NC = 2
NS = 16
L = 16
NW = NC * NS

CH = 128
GR = 64
IDXW = 128
SLAB = 1024
YROWS = 16


def _t2_body(f_ref, cg_ref, o_ref):
    o_ref[...] = jnp.dot(f_ref[...], cg_ref[...],
                         preferred_element_type=jnp.float32)


def _t2_call(features, cg2):
    n = features.shape[0]
    bn = 2000
    grid = n // bn
    return pl.pallas_call(
        _t2_body,
        grid=(grid,),
        in_specs=[
            pl.BlockSpec((bn, D_IN), lambda i: (i, 0)),
            pl.BlockSpec((D_IN, D_SH * D_OUT), lambda i: (0, 0)),
        ],
        out_specs=pl.BlockSpec((bn, D_SH * D_OUT), lambda i: (i, 0)),
        out_shape=jax.ShapeDtypeStruct((n, D_SH * D_OUT), jnp.float32),
    )(features, cg2)


def _ysw_body(r_ref, rad_ref, ys_ref, mus_ref, norm_ref, o_ref):
    basis = jnp.exp(-((rad_ref[...] - mus_ref[...]) ** 2))
    rs = jnp.sum(r_ref[...] * basis, axis=1, keepdims=True)
    o_ref[...] = ys_ref[...] * (rs * norm_ref[0, 0])


def _ysw_call(r_p, rad_p, ys_p, mus, norm2d):
    e_pad = r_p.shape[0]
    be = 4096
    grid = e_pad // be
    return pl.pallas_call(
        _ysw_body,
        grid=(grid,),
        in_specs=[
            pl.BlockSpec((be, D_RAD), lambda i: (i, 0)),
            pl.BlockSpec((be, 1), lambda i: (i, 0)),
            pl.BlockSpec((be, D_SH), lambda i: (i, 0)),
            pl.BlockSpec((1, D_RAD), lambda i: (0, 0)),
            pl.BlockSpec(memory_space=pltpu.SMEM),
        ],
        out_specs=pl.BlockSpec((be, D_SH), lambda i: (i, 0)),
        out_shape=jax.ShapeDtypeStruct((e_pad, D_SH), jnp.float32),
    )(r_p, rad_p, ys_p, mus, norm2d)


def _sc_body(n_pad, nslab, t2, idxb, idxa, ysw, zrows_hbm, out,
             idxb_v, idxa_v, ysw_v, rows_v, bout_v, f_sh, sem0, sem1):
    cid = lax.axis_index("c")
    sid = lax.axis_index("s")
    wid = sid * NC + cid

    rows_per_tile = n_pad // NS
    nbase = sid * rows_per_tile

    pltpu.sync_copy(zrows_hbm, f_sh.at[pl.ds(nbase, rows_per_tile)])
    plsc.subcore_barrier()

    epw = nslab * SLAB
    base_row = wid * (epw // IDXW)
    base_yrow = wid * (epw * D_SH // IDXW)
    lanes = lax.iota(jnp.int32, L)

    def slab(q, _):
        r0 = base_row + q * (SLAB // IDXW)
        pltpu.sync_copy(idxb.at[pl.ds(r0, SLAB // IDXW)], idxb_v)
        pltpu.sync_copy(idxa.at[pl.ds(r0, SLAB // IDXW)], idxa_v)
        for h in range(2):
            y0 = base_yrow + q * (SLAB * D_SH // IDXW) + h * YROWS
            pltpu.sync_copy(ysw.at[pl.ds(y0, YROWS)], ysw_v)
            for j in range(SLAB // (2 * CH)):
                r = h * (SLAB // (2 * CH)) + j
                pltpu.async_copy(t2.at[idxb_v.at[r]], rows_v, sem0).wait()

                def grp(g, _, j=j):
                    ri = lanes + g * L
                    wvecs = []
                    for s in range(D_SH):
                        flat = (ri + j * CH) * D_SH + s
                        wvecs.append(plsc.load_gather(
                            ysw_v, [lax.shift_right_logical(flat, 7),
                                    lax.bitwise_and(flat, IDXW - 1)]))
                    for o in range(D_OUT):
                        acc = None
                        for s in range(D_SH):
                            v = plsc.load_gather(
                                rows_v,
                                [ri, jnp.full((L,), s * D_OUT + o,
                                              jnp.int32)])
                            acc = (v * wvecs[s] if acc is None
                                   else acc + v * wvecs[s])
                        plsc.store_scatter(
                            bout_v, [ri, jnp.full((L,), o, jnp.int32)], acc)
                    return _
                lax.fori_loop(0, CH // L, grp, None)

                pltpu.sync_copy(bout_v, f_sh.at[idxa_v.at[r]], add=True)
        return _
    lax.fori_loop(0, nslab, slab, None)
    plsc.subcore_barrier()

    pltpu.sync_copy(f_sh.at[pl.ds(nbase, rows_per_tile)],
                    out.at[cid].at[pl.ds(nbase, rows_per_tile)])


def _sc_call(t2, idxb2, idxa2, ysw2, zrows, n_pad):
    e_pad = idxb2.shape[0] * IDXW
    nslab = e_pad // (NW * SLAB)
    mesh = plsc.VectorSubcoreMesh(core_axis_name="c", subcore_axis_name="s",
                                  num_cores=NC, num_subcores=NS)
    body = functools.partial(_sc_body, n_pad, nslab)
    f = pl.kernel(
        body,
        out_type=jax.ShapeDtypeStruct((NC, n_pad, D_OUT), jnp.float32),
        mesh=mesh,
        scratch_types=[
            pltpu.VMEM((SLAB // IDXW, IDXW), jnp.int32),
            pltpu.VMEM((SLAB // IDXW, IDXW), jnp.int32),
            pltpu.VMEM((YROWS, IDXW), jnp.float32),
            pltpu.VMEM((CH, IDXW), jnp.float32),
            pltpu.VMEM((CH, D_OUT), jnp.float32),
            pltpu.VMEM_SHARED((n_pad, D_OUT), jnp.float32),
            pltpu.SemaphoreType.DMA,
            pltpu.SemaphoreType.DMA,
        ],
        compiler_params=pltpu.CompilerParams(needs_layout_passes=False,
                                             use_tc_tiling_on_sc=False),
    )
    return f(t2, idxb2, idxa2, ysw2, zrows)


def _add_body(a_ref, b_ref, o_ref):
    o_ref[...] = a_ref[0] + b_ref[0]


def _add_call(partials, n):
    bn = 2000
    grid = n // bn
    return pl.pallas_call(
        _add_body,
        grid=(grid,),
        in_specs=[
            pl.BlockSpec((1, bn, D_OUT), lambda i: (0, i, 0)),
            pl.BlockSpec((1, bn, D_OUT), lambda i: (1, i, 0)),
        ],
        out_specs=pl.BlockSpec((bn, D_OUT), lambda i: (i, 0)),
        out_shape=jax.ShapeDtypeStruct((n, D_OUT), jnp.float32),
    )(partials, partials)


def kernel(features, radial_basis_function_coefficients, ab_p_to_a,
           ab_p_to_b, Ys, radii, cg, mus, norm_coef):
    n = features.shape[0]
    e = ab_p_to_a.shape[0]
    step = NW * SLAB
    e_pad = ((e + step - 1) // step) * step
    pad = e_pad - e
    n_pad = ((n + 8 * NS - 1) // (8 * NS)) * (8 * NS)

    cg2 = jnp.transpose(cg, (1, 2, 0)).reshape(D_IN, D_SH * D_OUT)
    r_p = jnp.pad(radial_basis_function_coefficients, ((0, pad), (0, 0)))
    rad_p = jnp.pad(radii, (0, pad)).reshape(e_pad, 1)
    ys_p = jnp.pad(Ys, ((0, pad), (0, 0)))
    idxb2 = jnp.pad(ab_p_to_b, (0, pad)).reshape(e_pad // IDXW, IDXW)
    idxa2 = jnp.pad(ab_p_to_a, (0, pad)).reshape(e_pad // IDXW, IDXW)
    mus2 = mus.reshape(1, D_RAD)
    norm2d = jnp.asarray(norm_coef, jnp.float32).reshape(1, 1)
    zrows = jnp.zeros((n_pad // NS, D_OUT), jnp.float32)

    t2 = _t2_call(features, cg2)
    ysw = _ysw_call(r_p, rad_p, ys_p, mus2, norm2d)
    ysw2 = ysw.reshape(e_pad * D_SH // IDXW, IDXW)
    partials = _sc_call(t2, idxb2, idxa2, ysw2, zrows, n_pad)
    return _add_call(partials, n)


# speedup vs baseline: 1.9742x; 1.2627x over previous
import functools

import jax
import jax.numpy as jnp
from jax import lax
from jax.experimental import pallas as pl
from jax.experimental.pallas import tpu as pltpu
from jax.experimental.pallas import tpu_sc as plsc

D_IN = 32
D_OUT = 32
D_SH = 4
D_RAD = 8

NC = 2
NS = 16
L = 16
NW = NC * NS

CH = 64
IW = 64
YW = 128
SLAB = 1024
YROWS = 32


def _t2_body(f_ref, cg_ref, o_ref):
    o_ref[...] = jnp.dot(f_ref[...], cg_ref[...],
                         preferred_element_type=jnp.float32)


def _t2_call(features, cg2):
    n = features.shape[0]
    bn = 2000
    grid = n // bn
    return pl.pallas_call(
        _t2_body,
        grid=(grid,),
        in_specs=[
            pl.BlockSpec((bn, D_IN), lambda i: (i, 0)),
            pl.BlockSpec((D_IN, D_SH * D_OUT), lambda i: (0, 0)),
        ],
        out_specs=pl.BlockSpec((bn, D_SH * D_OUT), lambda i: (i, 0)),
        out_shape=jax.ShapeDtypeStruct((n, D_SH * D_OUT), jnp.float32),
    )(features, cg2)


def _ysw_body(r_ref, rad_ref, ys_ref, mus_ref, norm_ref, o_ref):
    basis = jnp.exp(-((rad_ref[...] - mus_ref[...]) ** 2))
    rs = jnp.sum(r_ref[...] * basis, axis=1, keepdims=True)
    o_ref[...] = ys_ref[...] * (rs * norm_ref[0, 0])


def _ysw_call(r_p, rad_p, ys_p, mus, norm2d):
    e_pad = r_p.shape[0]
    be = 4096
    grid = e_pad // be
    return pl.pallas_call(
        _ysw_body,
        grid=(grid,),
        in_specs=[
            pl.BlockSpec((be, D_RAD), lambda i: (i, 0)),
            pl.BlockSpec((be, 1), lambda i: (i, 0)),
            pl.BlockSpec((be, D_SH), lambda i: (i, 0)),
            pl.BlockSpec((1, D_RAD), lambda i: (0, 0)),
            pl.BlockSpec(memory_space=pltpu.SMEM),
        ],
        out_specs=pl.BlockSpec((be, D_SH), lambda i: (i, 0)),
        out_shape=jax.ShapeDtypeStruct((e_pad, D_SH), jnp.float32),
    )(r_p, rad_p, ys_p, mus, norm2d)


def _sc_body(n_pad, nslab, t2, idxb, idxa, ysw, zrows_hbm, out,
             idxb_v, idxa_v, ysw_v, rows_v0, rows_v1, bout_v0, bout_v1,
             f_sh, gsem0, gsem1, ssem0, ssem1):
    cid = lax.axis_index("c")
    sid = lax.axis_index("s")
    wid = sid * NC + cid

    rows_per_tile = n_pad // NS
    nbase = sid * rows_per_tile

    pltpu.sync_copy(zrows_hbm, f_sh.at[pl.ds(nbase, rows_per_tile)])
    plsc.subcore_barrier()

    epw = nslab * SLAB
    base_row = wid * (epw // IW)
    base_yrow = wid * (epw * D_SH // YW)
    lanes = lax.iota(jnp.int32, L)
    rows_b = [rows_v0, rows_v1]
    bout_b = [bout_v0, bout_v1]
    gsem = [gsem0, gsem1]
    ssem = [ssem0, ssem1]
    ncr = SLAB // CH

    def slab(q, _):
        r0 = base_row + q * (SLAB // IW)
        pltpu.sync_copy(idxb.at[pl.ds(r0, SLAB // IW)], idxb_v)
        pltpu.sync_copy(idxa.at[pl.ds(r0, SLAB // IW)], idxa_v)
        y0 = base_yrow + q * YROWS
        pltpu.sync_copy(ysw.at[pl.ds(y0, YROWS)], ysw_v)

        gathers = [None] * ncr
        scatters = [None] * ncr
        gathers[0] = pltpu.async_copy(t2.at[idxb_v.at[0]], rows_b[0],
                                      gsem[0])
        for j in range(ncr):
            cur = j % 2
            gathers[j].wait()
            if j + 1 < ncr:
                gathers[j + 1] = pltpu.async_copy(
                    t2.at[idxb_v.at[j + 1]], rows_b[1 - cur], gsem[1 - cur])
            if j >= 2:
                scatters[j - 2].wait()

            def grp(g, _, j=j, cur=cur):
                ri = lanes + g * L
                wvecs = []
                for s in range(D_SH):
                    flat = (ri + j * CH) * D_SH + s
                    wvecs.append(plsc.load_gather(
                        ysw_v, [lax.shift_right_logical(flat, 7),
                                lax.bitwise_and(flat, YW - 1)]))
                for o in range(D_OUT):
                    acc = None
                    for s in range(D_SH):
                        v = plsc.load_gather(
                            rows_b[cur],
                            [ri, jnp.full((L,), s * D_OUT + o, jnp.int32)])
                        acc = (v * wvecs[s] if acc is None
                               else acc + v * wvecs[s])
                    plsc.store_scatter(
                        bout_b[cur], [ri, jnp.full((L,), o, jnp.int32)], acc)
                return _
            lax.fori_loop(0, CH // L, grp, None)

            scatters[j] = pltpu.async_copy(
                bout_b[cur], f_sh.at[idxa_v.at[j]], ssem[cur], add=True)
        scatters[ncr - 2].wait()
        scatters[ncr - 1].wait()
        return _
    lax.fori_loop(0, nslab, slab, None)
    plsc.subcore_barrier()

    pltpu.sync_copy(f_sh.at[pl.ds(nbase, rows_per_tile)],
                    out.at[cid].at[pl.ds(nbase, rows_per_tile)])


def _sc_call(t2, idxb2, idxa2, ysw2, zrows, n_pad):
    e_pad = idxb2.shape[0] * IW
    nslab = e_pad // (NW * SLAB)
    mesh = plsc.VectorSubcoreMesh(core_axis_name="c", subcore_axis_name="s",
                                  num_cores=NC, num_subcores=NS)
    body = functools.partial(_sc_body, n_pad, nslab)
    f = pl.kernel(
        body,
        out_type=jax.ShapeDtypeStruct((NC, n_pad, D_OUT), jnp.float32),
        mesh=mesh,
        scratch_types=[
            pltpu.VMEM((SLAB // IW, IW), jnp.int32),
            pltpu.VMEM((SLAB // IW, IW), jnp.int32),
            pltpu.VMEM((YROWS, YW), jnp.float32),
            pltpu.VMEM((CH, D_SH * D_OUT), jnp.float32),
            pltpu.VMEM((CH, D_SH * D_OUT), jnp.float32),
            pltpu.VMEM((CH, D_OUT), jnp.float32),
            pltpu.VMEM((CH, D_OUT), jnp.float32),
            pltpu.VMEM_SHARED((n_pad, D_OUT), jnp.float32),
            pltpu.SemaphoreType.DMA,
            pltpu.SemaphoreType.DMA,
            pltpu.SemaphoreType.DMA,
            pltpu.SemaphoreType.DMA,
        ],
        compiler_params=pltpu.CompilerParams(needs_layout_passes=False,
                                             use_tc_tiling_on_sc=False),
    )
    return f(t2, idxb2, idxa2, ysw2, zrows)


def _add_body(a_ref, b_ref, o_ref):
    o_ref[...] = a_ref[0] + b_ref[0]


def _add_call(partials, n):
    bn = 2000
    grid = n // bn
    return pl.pallas_call(
        _add_body,
        grid=(grid,),
        in_specs=[
            pl.BlockSpec((1, bn, D_OUT), lambda i: (0, i, 0)),
            pl.BlockSpec((1, bn, D_OUT), lambda i: (1, i, 0)),
        ],
        out_specs=pl.BlockSpec((bn, D_OUT), lambda i: (i, 0)),
        out_shape=jax.ShapeDtypeStruct((n, D_OUT), jnp.float32),
    )(partials, partials)


def kernel(features, radial_basis_function_coefficients, ab_p_to_a,
           ab_p_to_b, Ys, radii, cg, mus, norm_coef):
    n = features.shape[0]
    e = ab_p_to_a.shape[0]
    step = NW * SLAB
    e_pad = ((e + step - 1) // step) * step
    pad = e_pad - e
    n_pad = ((n + 8 * NS - 1) // (8 * NS)) * (8 * NS)

    cg2 = jnp.transpose(cg, (1, 2, 0)).reshape(D_IN, D_SH * D_OUT)
    r_p = jnp.pad(radial_basis_function_coefficients, ((0, pad), (0, 0)))
    rad_p = jnp.pad(radii, (0, pad)).reshape(e_pad, 1)
    ys_p = jnp.pad(Ys, ((0, pad), (0, 0)))
    idxb2 = jnp.pad(ab_p_to_b, (0, pad)).reshape(e_pad // IW, IW)
    idxa2 = jnp.pad(ab_p_to_a, (0, pad)).reshape(e_pad // IW, IW)
    mus2 = mus.reshape(1, D_RAD)
    norm2d = jnp.asarray(norm_coef, jnp.float32).reshape(1, 1)
    zrows = jnp.zeros((n_pad // NS, D_OUT), jnp.float32)

    t2 = _t2_call(features, cg2)
    ysw = _ysw_call(r_p, rad_p, ys_p, mus2, norm2d)
    ysw2 = ysw.reshape(e_pad * D_SH // YW, YW)
    partials = _sc_call(t2, idxb2, idxa2, ysw2, zrows, n_pad)
    return _add_call(partials, n)


# speedup vs baseline: 2.4979x; 1.2653x over previous
import functools

import jax
import jax.numpy as jnp
from jax import lax
from jax.experimental import pallas as pl
from jax.experimental.pallas import tpu as pltpu
from jax.experimental.pallas import tpu_sc as plsc

D_IN = 32
D_OUT = 32
D_SH = 4
D_RAD = 8

NC = 2
NS = 16
L = 16
NW = NC * NS

CH = 64
IW = 64
YW = 128
SLAB = 1024
YROWS = 32


def _t2_body(f_ref, cg_ref, o_ref):
    o_ref[...] = jnp.dot(f_ref[...], cg_ref[...],
                         preferred_element_type=jnp.float32)


def _t2_call(features, cg2):
    n = features.shape[0]
    bn = 2000
    grid = n // bn
    return pl.pallas_call(
        _t2_body,
        grid=(grid,),
        in_specs=[
            pl.BlockSpec((bn, D_IN), lambda i: (i, 0)),
            pl.BlockSpec((D_IN, D_SH * D_OUT), lambda i: (0, 0)),
        ],
        out_specs=pl.BlockSpec((bn, D_SH * D_OUT), lambda i: (i, 0)),
        out_shape=jax.ShapeDtypeStruct((n, D_SH * D_OUT), jnp.float32),
    )(features, cg2)


def _ysw_body(r_ref, rad_ref, ys_ref, mus_ref, norm_ref, o_ref):
    basis = jnp.exp(-((rad_ref[...] - mus_ref[...]) ** 2))
    rs = jnp.sum(r_ref[...] * basis, axis=1, keepdims=True)
    o_ref[...] = ys_ref[...] * (rs * norm_ref[0, 0])


def _ysw_call(r_p, rad_p, ys_p, mus, norm2d):
    e_pad = r_p.shape[0]
    be = 4096
    grid = e_pad // be
    return pl.pallas_call(
        _ysw_body,
        grid=(grid,),
        in_specs=[
            pl.BlockSpec((be, D_RAD), lambda i: (i, 0)),
            pl.BlockSpec((be, 1), lambda i: (i, 0)),
            pl.BlockSpec((be, D_SH), lambda i: (i, 0)),
            pl.BlockSpec((1, D_RAD), lambda i: (0, 0)),
            pl.BlockSpec(memory_space=pltpu.SMEM),
        ],
        out_specs=pl.BlockSpec((be, D_SH), lambda i: (i, 0)),
        out_shape=jax.ShapeDtypeStruct((e_pad, D_SH), jnp.float32),
    )(r_p, rad_p, ys_p, mus, norm2d)


def _sc_body(n_pad, nslab, t2, idxb, idxa, ysw, zrows_hbm, out,
             idxb_v, idxa_v, ysw_v, rows_v0, rows_v1, bout_v0, bout_v1,
             f_sh, gsem0, gsem1, ssem0, ssem1):
    cid = lax.axis_index("c")
    sid = lax.axis_index("s")
    wid = sid * NC + cid

    rows_per_tile = n_pad // NS
    nbase = sid * rows_per_tile

    pltpu.sync_copy(zrows_hbm, f_sh.at[pl.ds(nbase, rows_per_tile)])
    plsc.subcore_barrier()

    epw = nslab * SLAB
    base_row = wid * (epw // IW)
    base_yrow = wid * (epw * D_SH // YW)
    lanes = lax.iota(jnp.int32, L)
    rows_b = [rows_v0, rows_v1]
    bout_b = [bout_v0, bout_v1]
    gsem = [gsem0, gsem1]
    ssem = [ssem0, ssem1]
    ncr = SLAB // CH

    def slab(q, _):
        r0 = base_row + q * (SLAB // IW)
        pltpu.sync_copy(idxb.at[pl.ds(r0, SLAB // IW)], idxb_v)
        pltpu.sync_copy(idxa.at[pl.ds(r0, SLAB // IW)], idxa_v)
        y0 = base_yrow + q * YROWS
        pltpu.sync_copy(ysw.at[pl.ds(y0, YROWS)], ysw_v)

        gathers = [None] * ncr
        scatters = [None] * ncr
        gathers[0] = pltpu.async_copy(t2.at[idxb_v.at[0]], rows_b[0],
                                      gsem[0])
        for j in range(ncr):
            cur = j % 2
            gathers[j].wait()
            if j + 1 < ncr:
                gathers[j + 1] = pltpu.async_copy(
                    t2.at[idxb_v.at[j + 1]], rows_b[1 - cur], gsem[1 - cur])
            if j >= 2:
                scatters[j - 2].wait()

            def edge(p, _, j=j, cur=cur):
                fbase = (j * CH + p) * D_SH
                row = lax.shift_right_logical(fbase, 7)
                lane = lax.bitwise_and(fbase, YW - 1)
                zeros = jnp.zeros((L,), jnp.int32)
                w = [plsc.load_gather(ysw_v, [zeros + row, zeros + lane + s])
                     for s in range(D_SH)]
                for h in range(D_OUT // L):
                    acc = None
                    for s in range(D_SH):
                        v = rows_b[cur][p, pl.ds(s * D_OUT + h * L, L)]
                        acc = v * w[s] if acc is None else acc + v * w[s]
                    bout_b[cur][p, pl.ds(h * L, L)] = acc
                return _
            lax.fori_loop(0, CH, edge, None, unroll=4)

            scatters[j] = pltpu.async_copy(
                bout_b[cur], f_sh.at[idxa_v.at[j]], ssem[cur], add=True)
        scatters[ncr - 2].wait()
        scatters[ncr - 1].wait()
        return _
    lax.fori_loop(0, nslab, slab, None)
    plsc.subcore_barrier()

    pltpu.sync_copy(f_sh.at[pl.ds(nbase, rows_per_tile)],
                    out.at[cid].at[pl.ds(nbase, rows_per_tile)])


def _sc_call(t2, idxb2, idxa2, ysw2, zrows, n_pad):
    e_pad = idxb2.shape[0] * IW
    nslab = e_pad // (NW * SLAB)
    mesh = plsc.VectorSubcoreMesh(core_axis_name="c", subcore_axis_name="s",
                                  num_cores=NC, num_subcores=NS)
    body = functools.partial(_sc_body, n_pad, nslab)
    f = pl.kernel(
        body,
        out_type=jax.ShapeDtypeStruct((NC, n_pad, D_OUT), jnp.float32),
        mesh=mesh,
        scratch_types=[
            pltpu.VMEM((SLAB // IW, IW), jnp.int32),
            pltpu.VMEM((SLAB // IW, IW), jnp.int32),
            pltpu.VMEM((YROWS, YW), jnp.float32),
            pltpu.VMEM((CH, D_SH * D_OUT), jnp.float32),
            pltpu.VMEM((CH, D_SH * D_OUT), jnp.float32),
            pltpu.VMEM((CH, D_OUT), jnp.float32),
            pltpu.VMEM((CH, D_OUT), jnp.float32),
            pltpu.VMEM_SHARED((n_pad, D_OUT), jnp.float32),
            pltpu.SemaphoreType.DMA,
            pltpu.SemaphoreType.DMA,
            pltpu.SemaphoreType.DMA,
            pltpu.SemaphoreType.DMA,
        ],
        compiler_params=pltpu.CompilerParams(needs_layout_passes=False,
                                             use_tc_tiling_on_sc=False),
    )
    return f(t2, idxb2, idxa2, ysw2, zrows)


def _add_body(a_ref, b_ref, o_ref):
    o_ref[...] = a_ref[0] + b_ref[0]


def _add_call(partials, n):
    bn = 2000
    grid = n // bn
    return pl.pallas_call(
        _add_body,
        grid=(grid,),
        in_specs=[
            pl.BlockSpec((1, bn, D_OUT), lambda i: (0, i, 0)),
            pl.BlockSpec((1, bn, D_OUT), lambda i: (1, i, 0)),
        ],
        out_specs=pl.BlockSpec((bn, D_OUT), lambda i: (i, 0)),
        out_shape=jax.ShapeDtypeStruct((n, D_OUT), jnp.float32),
    )(partials, partials)


def kernel(features, radial_basis_function_coefficients, ab_p_to_a,
           ab_p_to_b, Ys, radii, cg, mus, norm_coef):
    n = features.shape[0]
    e = ab_p_to_a.shape[0]
    step = NW * SLAB
    e_pad = ((e + step - 1) // step) * step
    pad = e_pad - e
    n_pad = ((n + 8 * NS - 1) // (8 * NS)) * (8 * NS)

    cg2 = jnp.transpose(cg, (1, 2, 0)).reshape(D_IN, D_SH * D_OUT)
    r_p = jnp.pad(radial_basis_function_coefficients, ((0, pad), (0, 0)))
    rad_p = jnp.pad(radii, (0, pad)).reshape(e_pad, 1)
    ys_p = jnp.pad(Ys, ((0, pad), (0, 0)))
    idxb2 = jnp.pad(ab_p_to_b, (0, pad)).reshape(e_pad // IW, IW)
    idxa2 = jnp.pad(ab_p_to_a, (0, pad)).reshape(e_pad // IW, IW)
    mus2 = mus.reshape(1, D_RAD)
    norm2d = jnp.asarray(norm_coef, jnp.float32).reshape(1, 1)
    zrows = jnp.zeros((n_pad // NS, D_OUT), jnp.float32)

    t2 = _t2_call(features, cg2)
    ysw = _ysw_call(r_p, rad_p, ys_p, mus2, norm2d)
    ysw2 = ysw.reshape(e_pad * D_SH // YW, YW)
    partials = _sc_call(t2, idxb2, idxa2, ysw2, zrows, n_pad)
    return _add_call(partials, n)
